# Initial kernel scaffold; baseline (speedup 1.0000x reference)
import jax
import jax.numpy as jnp
from jax.experimental import pallas as pl


def kernel(x, edge_index, SP, W):
    raise NotImplementedError("write your pallas kernel here")


# speedup vs baseline: 1.6661x; 1.6661x over previous
import functools

import jax
import jax.numpy as jnp
from jax.experimental import pallas as pl
from jax.experimental.pallas import tpu as pltpu

N_NODES = 10000
NINP = 128
NOUT = 128
K = 4


def _matmul_body(x_ref, w_ref, o_ref):
    o_ref[...] = jnp.dot(x_ref[...], w_ref[...],
                         preferred_element_type=jnp.float32)


def _matmul(x, w, block_rows=2000):
    m, k = x.shape
    _, n = w.shape
    return pl.pallas_call(
        _matmul_body,
        grid=(m // block_rows,),
        in_specs=[
            pl.BlockSpec((block_rows, k), lambda i: (i, 0)),
            pl.BlockSpec((k, n), lambda i: (0, 0)),
        ],
        out_specs=pl.BlockSpec((block_rows, n), lambda i: (i, 0)),
        out_shape=jax.ShapeDtypeStruct((m, n), jnp.float32),
    )(x, w)


def _relu_body(h_ref, o_ref):
    o_ref[...] = jnp.maximum(h_ref[...], 0.0)


def _relu(h, block_rows=2000):
    m, n = h.shape
    return pl.pallas_call(
        _relu_body,
        grid=(m // block_rows,),
        in_specs=[pl.BlockSpec((block_rows, n), lambda i: (i, 0))],
        out_specs=pl.BlockSpec((block_rows, n), lambda i: (i, 0)),
        out_shape=jax.ShapeDtypeStruct((m, n), jnp.float32),
    )(h)


def kernel(x, edge_index, SP, W):
    src = edge_index[0]
    dst = edge_index[1]
    wcat = jnp.transpose(W, (1, 0, 2)).reshape(NINP, K * NOUT)
    z = _matmul(x, wcat)
    zg = z.reshape(N_NODES, K, NOUT)[src]
    msg = jnp.einsum('ek,ekc->ec', SP, zg)
    h = jax.ops.segment_sum(msg, dst, num_segments=N_NODES)
    return _relu(h)
